# baseline (speedup 1.0000x reference)
import functools

import jax
import jax.numpy as jnp
import numpy as np
from jax import lax
from jax.experimental import pallas as pl
from jax.experimental.pallas import tpu as pltpu
from jax.experimental.pallas import tpu_sc as plsc

N = 10000
D = 128
H = 128
C = 16
E = 320000
NC, NS, L = 2, 16, 16
NW = NC * NS
NPAD = 10240
RPT = NPAD // NS
EPAD = 327680
PADN = EPAD - E
K = 128
BN = 1000


def _mesh():
    return plsc.VectorSubcoreMesh(
        core_axis_name="c", subcore_axis_name="s", num_cores=NC, num_subcores=NS
    )


def _deg_call(src2d, dst2d):
    nch = EPAD // K // NS
    nhalf = nch // 2

    @functools.partial(
        pl.kernel,
        out_type=jax.ShapeDtypeStruct((2, NPAD), jnp.float32),
        mesh=_mesh(),
        scratch_types=[
            pltpu.VMEM((nhalf, K), jnp.int32),
            pltpu.VMEM((K,), jnp.float32),
            pltpu.VMEM((RPT,), jnp.float32),
            pltpu.VMEM_SHARED((NPAD,), jnp.float32),
        ] + [pltpu.SemaphoreType.DMA] * 8,
        compiler_params=pltpu.CompilerParams(use_tc_tiling_on_sc=False),
    )
    def deg_kernel(src_ref, dst_ref, out_ref, idx2, ones, zb, acc, *sems):
        c = lax.axis_index("c")
        s = lax.axis_index("s")
        one16 = jnp.ones((L,), jnp.float32)
        zero16 = jnp.zeros((L,), jnp.float32)
        for j in range(K // L):
            ones[pl.ds(j * L, L)] = one16

        def zb_body(j, carry):
            zb[pl.ds(pl.multiple_of(j * L, 8), L)] = zero16
            return carry

        lax.fori_loop(0, RPT // L, zb_body, 0)
        pltpu.sync_copy(zb, acc.at[pl.ds(pl.multiple_of(s * RPT, 8), RPT)])
        plsc.subcore_barrier()
        nb = len(sems)

        def run(ref):
            for half in range(2):
                pltpu.sync_copy(ref.at[pl.ds(s * nch + half * nhalf, nhalf)], idx2)
                for b in range(nb):
                    pltpu.async_copy(ones, acc.at[idx2.at[b]], sems[b], add=True)

                @pl.loop(0, nhalf, step=nb)
                def _(g0):
                    for b in range(nb):
                        i = g0 + b
                        pltpu.make_async_copy(ones, acc.at[idx2.at[0]], sems[b]).wait()

                        @pl.when(i + nb < nhalf)
                        def _():
                            pltpu.async_copy(
                                ones, acc.at[idx2.at[i + nb]], sems[b], add=True
                            )

        @pl.when(c == 0)
        def _():
            run(src_ref)

        @pl.when(c == 1)
        def _():
            run(dst_ref)

        plsc.subcore_barrier()
        st = pl.multiple_of(s * RPT, 8)
        pltpu.sync_copy(acc.at[pl.ds(st, RPT)], out_ref.at[c, pl.ds(st, RPT)])

    return deg_kernel(src2d, dst2d)


NB = 5
ZR = 40


def _zero_acc(zb, acc, s, cpr):
    zero16 = jnp.zeros((L,), jnp.float32)

    def zb_body(j, carry):
        zb[j // cpr, pl.ds((j % cpr) * L, L)] = zero16
        return carry

    lax.fori_loop(0, ZR * cpr, zb_body, 0)

    def zc_body(j, carry):
        pltpu.sync_copy(zb, acc.at[pl.ds(pl.multiple_of(s * RPT + j * ZR, 8), ZR)])
        return carry

    lax.fori_loop(0, RPT // ZR, zc_body, 0)


def _agg_ring(tab, sidx2, didx2, rows, acc, gsems, ssems, nch):
    nb = len(gsems)
    for b in range(nb):
        pltpu.async_copy(tab.at[sidx2.at[b]], rows.at[b], gsems[b])

    @pl.loop(0, nch, step=nb)
    def _(g0):
        for b in range(nb):
            i = g0 + b
            pltpu.make_async_copy(tab.at[sidx2.at[0]], rows.at[b], gsems[b]).wait()
            pltpu.async_copy(rows.at[b], acc.at[didx2.at[i]], ssems[b], add=True)

            @pl.when(i + nb < nch)
            def _():
                pltpu.make_async_copy(rows.at[b], acc.at[didx2.at[0]], ssems[b]).wait()
                pltpu.async_copy(tab.at[sidx2.at[i + nb]], rows.at[b], gsems[b])

    for b in range(nb):
        pltpu.make_async_copy(rows.at[b], acc.at[didx2.at[0]], ssems[b]).wait()


def _agg_split_call(h2lay, src_pg, dst_pd):
    F = H // NC
    nch = EPAD // K // NS
    nhalf = nch // 2
    nb1 = 8
    cpr = F // L

    @functools.partial(
        pl.kernel,
        out_type=jax.ShapeDtypeStruct((NC, NPAD, F), jnp.float32),
        mesh=_mesh(),
        scratch_types=[
            pltpu.VMEM((nhalf, K), jnp.int32),
            pltpu.VMEM((nhalf, K), jnp.int32),
            pltpu.VMEM((nb1, K, F), jnp.float32),
            pltpu.VMEM((ZR, F), jnp.float32),
            pltpu.VMEM_SHARED((NPAD, F), jnp.float32),
        ] + [pltpu.SemaphoreType.DMA] * (2 * nb1),
        compiler_params=pltpu.CompilerParams(use_tc_tiling_on_sc=False),
    )
    def agg_kernel(h_ref, src_ref, dst_ref, out_ref, sidx2, didx2, rows, zb, acc, *sems):
        c = lax.axis_index("c")
        s = lax.axis_index("s")
        _zero_acc(zb, acc, s, cpr)
        plsc.subcore_barrier()
        for half in range(2):
            base = s * nch + half * nhalf
            pltpu.sync_copy(src_ref.at[pl.ds(base, nhalf)], sidx2)
            pltpu.sync_copy(dst_ref.at[pl.ds(base, nhalf)], didx2)
            _agg_ring(h_ref.at[c], sidx2, didx2, rows, acc, sems[:nb1], sems[nb1:], nhalf)
        plsc.subcore_barrier()
        st = pl.multiple_of(s * RPT, 8)
        pltpu.sync_copy(acc.at[pl.ds(st, RPT)], out_ref.at[c, pl.ds(st, RPT)])

    return agg_kernel(h2lay, src_pg, dst_pd)


def _agg2_call(h2, src_pg, dst_pd):
    F = C
    nch = EPAD // K // NW
    nb2 = 8
    cpr = F // L

    @functools.partial(
        pl.kernel,
        out_type=jax.ShapeDtypeStruct((NC, NPAD, F), jnp.float32),
        mesh=_mesh(),
        scratch_types=[
            pltpu.VMEM((nch, K), jnp.int32),
            pltpu.VMEM((nch, K), jnp.int32),
            pltpu.VMEM((nb2, K, F), jnp.float32),
            pltpu.VMEM((ZR, F), jnp.float32),
            pltpu.VMEM_SHARED((NPAD, F), jnp.float32),
        ] + [pltpu.SemaphoreType.DMA] * (2 * nb2),
        compiler_params=pltpu.CompilerParams(use_tc_tiling_on_sc=False),
    )
    def agg_kernel(h_ref, src_ref, dst_ref, out_ref, sidx2, didx2, rows, zb, acc, *sems):
        c = lax.axis_index("c")
        s = lax.axis_index("s")
        w = s * NC + c
        _zero_acc(zb, acc, s, cpr)
        pltpu.sync_copy(src_ref.at[pl.ds(w * nch, nch)], sidx2)
        pltpu.sync_copy(dst_ref.at[pl.ds(w * nch, nch)], didx2)
        plsc.subcore_barrier()
        _agg_ring(h_ref, sidx2, didx2, rows, acc, sems[:nb2], sems[nb2:], nch)
        plsc.subcore_barrier()
        st = pl.multiple_of(s * RPT, 8)
        pltpu.sync_copy(acc.at[pl.ds(st, RPT)], out_ref.at[c, pl.ds(st, RPT)])

    return agg_kernel(h2, src_pg, dst_pd)


def _norm_from(deg_row):
    return jnp.where(deg_row > 0.0, lax.rsqrt(deg_row), 0.0)


def _norm_src_from(deg_ref):
    i = pl.program_id(0)
    ids = lax.broadcasted_iota(jnp.int32, (BN,), 0) + i * BN
    d = deg_ref[:, 0] - jnp.where(ids < PADN, 1.0, 0.0)
    return _norm_from(d)


def _mm1_call(x, W1, degT):
    F = H // NC

    def body(x_ref, w_ref, deg_ref, o_ref):
        ns = _norm_src_from(deg_ref)
        y = jnp.dot(x_ref[...], w_ref[...], preferred_element_type=jnp.float32)
        y = y * ns[:, None]
        o_ref[0] = y[:, :F]
        o_ref[1] = y[:, F:]

    return pl.pallas_call(
        body,
        grid=(N // BN,),
        in_specs=[
            pl.BlockSpec((BN, D), lambda i: (i, 0)),
            pl.BlockSpec((D, H), lambda i: (0, 0)),
            pl.BlockSpec((BN, 2), lambda i: (i, 0)),
        ],
        out_specs=pl.BlockSpec((2, BN, F), lambda i: (0, i, 0)),
        out_shape=jax.ShapeDtypeStruct((2, N, F), jnp.float32),
    )(x, W1, degT)


def _mid_call(parts, degT, b1, W2):
    F = H // NC

    def body(p_ref, deg_ref, b1_ref, w2_ref, o_ref):
        agg = jnp.concatenate([p_ref[0], p_ref[1]], axis=1)
        nd = _norm_from(deg_ref[:, 1])
        ns = _norm_src_from(deg_ref)
        t = jnp.maximum(agg * nd[:, None] + b1_ref[...][None, :], 0.0) * ns[:, None]
        o_ref[...] = jnp.dot(t, w2_ref[...], preferred_element_type=jnp.float32)

    return pl.pallas_call(
        body,
        grid=(N // BN,),
        in_specs=[
            pl.BlockSpec((2, BN, F), lambda i: (0, i, 0)),
            pl.BlockSpec((BN, 2), lambda i: (i, 0)),
            pl.BlockSpec((H,), lambda i: (0,)),
            pl.BlockSpec((H, C), lambda i: (0, 0)),
        ],
        out_specs=pl.BlockSpec((BN, C), lambda i: (i, 0)),
        out_shape=jax.ShapeDtypeStruct((N, C), jnp.float32),
    )(parts, degT, b1, W2)


_PAD_DST = np.asarray(N + (np.arange(PADN) % 16), np.int32)
_PAD_SRC = np.asarray(np.arange(PADN), np.int32)


def kernel(x, edge_index, W1, b1, W2, b2):
    src = edge_index[0]
    dst = edge_index[1]
    dst_pd = jnp.concatenate([dst, jnp.asarray(_PAD_DST)]).reshape(EPAD // K, K)
    src_pg = jnp.concatenate([src, jnp.asarray(_PAD_SRC)]).reshape(EPAD // K, K)

    deg = _deg_call(src_pg, dst_pd)
    degT = deg.T
    h = _mm1_call(x, W1, degT)
    parts1 = _agg_split_call(h, src_pg, dst_pd)
    h2 = _mid_call(parts1, degT, b1, W2)
    parts2 = _agg2_call(h2, src_pg, dst_pd)
    nd = _norm_from(deg[1])
    out_full = (parts2[0] + parts2[1]) * nd[:, None] + b2[None, :]
    return out_full[:N]


# speedup vs baseline: 1.0141x; 1.0141x over previous
import functools

import jax
import jax.numpy as jnp
import numpy as np
from jax import lax
from jax.experimental import pallas as pl
from jax.experimental.pallas import tpu as pltpu
from jax.experimental.pallas import tpu_sc as plsc

N = 10000
D = 128
H = 128
C = 16
E = 320000
NC, NS, L = 2, 16, 16
NW = NC * NS
NPAD = 10240
RPT = NPAD // NS
EPAD = 327680
PADN = EPAD - E
K = 128
BN = 1000


def _mesh():
    return plsc.VectorSubcoreMesh(
        core_axis_name="c", subcore_axis_name="s", num_cores=NC, num_subcores=NS
    )


def _deg_call(src2d, dst2d):
    nch = EPAD // K // NS
    nhalf = nch // 2

    @functools.partial(
        pl.kernel,
        out_type=jax.ShapeDtypeStruct((2, NPAD), jnp.float32),
        mesh=_mesh(),
        scratch_types=[
            pltpu.VMEM((nhalf, K), jnp.int32),
            pltpu.VMEM((K,), jnp.float32),
            pltpu.VMEM((RPT,), jnp.float32),
            pltpu.VMEM_SHARED((NPAD,), jnp.float32),
        ] + [pltpu.SemaphoreType.DMA] * 8,
        compiler_params=pltpu.CompilerParams(use_tc_tiling_on_sc=False),
    )
    def deg_kernel(src_ref, dst_ref, out_ref, idx2, ones, zb, acc, *sems):
        c = lax.axis_index("c")
        s = lax.axis_index("s")
        one16 = jnp.ones((L,), jnp.float32)
        zero16 = jnp.zeros((L,), jnp.float32)
        for j in range(K // L):
            ones[pl.ds(j * L, L)] = one16

        def zb_body(j, carry):
            zb[pl.ds(pl.multiple_of(j * L, 8), L)] = zero16
            return carry

        lax.fori_loop(0, RPT // L, zb_body, 0)
        pltpu.sync_copy(zb, acc.at[pl.ds(pl.multiple_of(s * RPT, 8), RPT)])
        plsc.subcore_barrier()
        nb = len(sems)

        def run(ref):
            for half in range(2):
                pltpu.sync_copy(ref.at[pl.ds(s * nch + half * nhalf, nhalf)], idx2)
                for b in range(nb):
                    pltpu.async_copy(ones, acc.at[idx2.at[b]], sems[b], add=True)

                @pl.loop(0, nhalf, step=nb)
                def _(g0):
                    for b in range(nb):
                        i = g0 + b
                        pltpu.make_async_copy(ones, acc.at[idx2.at[0]], sems[b]).wait()

                        @pl.when(i + nb < nhalf)
                        def _():
                            pltpu.async_copy(
                                ones, acc.at[idx2.at[i + nb]], sems[b], add=True
                            )

        @pl.when(c == 0)
        def _():
            run(src_ref)

        @pl.when(c == 1)
        def _():
            run(dst_ref)

        plsc.subcore_barrier()
        st = pl.multiple_of(s * RPT, 8)
        pltpu.sync_copy(acc.at[pl.ds(st, RPT)], out_ref.at[c, pl.ds(st, RPT)])

    return deg_kernel(src2d, dst2d)


NB = 5
ZR = 40


def _zero_acc(zb, acc, s, cpr):
    zero16 = jnp.zeros((L,), jnp.float32)

    def zb_body(j, carry):
        zb[j // cpr, pl.ds((j % cpr) * L, L)] = zero16
        return carry

    lax.fori_loop(0, ZR * cpr, zb_body, 0)

    def zc_body(j, carry):
        pltpu.sync_copy(zb, acc.at[pl.ds(pl.multiple_of(s * RPT + j * ZR, 8), ZR)])
        return carry

    lax.fori_loop(0, RPT // ZR, zc_body, 0)


def _agg_ring(tab, sidx2, didx2, rows, acc, gsems, ssems, nch):
    nb = len(gsems)
    for b in range(nb):
        pltpu.async_copy(tab.at[sidx2.at[b]], rows.at[b], gsems[b])

    @pl.loop(0, nch, step=nb)
    def _(g0):
        for b in range(nb):
            i = g0 + b
            pltpu.make_async_copy(tab.at[sidx2.at[0]], rows.at[b], gsems[b]).wait()
            pltpu.async_copy(rows.at[b], acc.at[didx2.at[i]], ssems[b], add=True)

            @pl.when(i + nb < nch)
            def _():
                pltpu.make_async_copy(rows.at[b], acc.at[didx2.at[0]], ssems[b]).wait()
                pltpu.async_copy(tab.at[sidx2.at[i + nb]], rows.at[b], gsems[b])

    for b in range(nb):
        pltpu.make_async_copy(rows.at[b], acc.at[didx2.at[0]], ssems[b]).wait()


def _agg_split_call(h2lay, src_pg, dst_pd):
    F = H // NC
    nch = EPAD // K // NS
    nhalf = nch // 2
    nb1 = 5
    cpr = F // L

    @functools.partial(
        pl.kernel,
        out_type=jax.ShapeDtypeStruct((NC, NPAD, F), jnp.float32),
        mesh=_mesh(),
        scratch_types=[
            pltpu.VMEM((nhalf, K), jnp.int32),
            pltpu.VMEM((nhalf, K), jnp.int32),
            pltpu.VMEM((nb1, K, F), jnp.float32),
            pltpu.VMEM((ZR, F), jnp.float32),
            pltpu.VMEM_SHARED((NPAD, F), jnp.float32),
        ] + [pltpu.SemaphoreType.DMA] * (2 * nb1),
        compiler_params=pltpu.CompilerParams(use_tc_tiling_on_sc=False),
    )
    def agg_kernel(h_ref, src_ref, dst_ref, out_ref, sidx2, didx2, rows, zb, acc, *sems):
        c = lax.axis_index("c")
        s = lax.axis_index("s")
        _zero_acc(zb, acc, s, cpr)
        plsc.subcore_barrier()
        for half in range(2):
            base = s * nch + half * nhalf
            pltpu.sync_copy(src_ref.at[pl.ds(base, nhalf)], sidx2)
            pltpu.sync_copy(dst_ref.at[pl.ds(base, nhalf)], didx2)
            _agg_ring(h_ref.at[c], sidx2, didx2, rows, acc, sems[:nb1], sems[nb1:], nhalf)
        plsc.subcore_barrier()
        st = pl.multiple_of(s * RPT, 8)
        pltpu.sync_copy(acc.at[pl.ds(st, RPT)], out_ref.at[c, pl.ds(st, RPT)])

    return agg_kernel(h2lay, src_pg, dst_pd)


def _agg2_call(h2, src_pg, dst_pd):
    F = C
    nch = EPAD // K // NW
    nb2 = 8
    cpr = F // L

    @functools.partial(
        pl.kernel,
        out_type=jax.ShapeDtypeStruct((NC, NPAD, F), jnp.float32),
        mesh=_mesh(),
        scratch_types=[
            pltpu.VMEM((nch, K), jnp.int32),
            pltpu.VMEM((nch, K), jnp.int32),
            pltpu.VMEM((nb2, K, F), jnp.float32),
            pltpu.VMEM((ZR, F), jnp.float32),
            pltpu.VMEM_SHARED((NPAD, F), jnp.float32),
        ] + [pltpu.SemaphoreType.DMA] * (2 * nb2),
        compiler_params=pltpu.CompilerParams(use_tc_tiling_on_sc=False),
    )
    def agg_kernel(h_ref, src_ref, dst_ref, out_ref, sidx2, didx2, rows, zb, acc, *sems):
        c = lax.axis_index("c")
        s = lax.axis_index("s")
        w = s * NC + c
        _zero_acc(zb, acc, s, cpr)
        pltpu.sync_copy(src_ref.at[pl.ds(w * nch, nch)], sidx2)
        pltpu.sync_copy(dst_ref.at[pl.ds(w * nch, nch)], didx2)
        plsc.subcore_barrier()
        _agg_ring(h_ref, sidx2, didx2, rows, acc, sems[:nb2], sems[nb2:], nch)
        plsc.subcore_barrier()
        st = pl.multiple_of(s * RPT, 8)
        pltpu.sync_copy(acc.at[pl.ds(st, RPT)], out_ref.at[c, pl.ds(st, RPT)])

    return agg_kernel(h2, src_pg, dst_pd)


def _norm_from(deg_row):
    return jnp.where(deg_row > 0.0, lax.rsqrt(deg_row), 0.0)


def _norm_src_from(deg_ref):
    i = pl.program_id(0)
    ids = lax.broadcasted_iota(jnp.int32, (BN,), 0) + i * BN
    d = deg_ref[:, 0] - jnp.where(ids < PADN, 1.0, 0.0)
    return _norm_from(d)


def _mm1_call(x, W1, degT):
    F = H // NC

    def body(x_ref, w_ref, deg_ref, o_ref):
        ns = _norm_src_from(deg_ref)
        y = jnp.dot(x_ref[...], w_ref[...], preferred_element_type=jnp.float32)
        y = y * ns[:, None]
        o_ref[0] = y[:, :F]
        o_ref[1] = y[:, F:]

    return pl.pallas_call(
        body,
        grid=(N // BN,),
        in_specs=[
            pl.BlockSpec((BN, D), lambda i: (i, 0)),
            pl.BlockSpec((D, H), lambda i: (0, 0)),
            pl.BlockSpec((BN, 2), lambda i: (i, 0)),
        ],
        out_specs=pl.BlockSpec((2, BN, F), lambda i: (0, i, 0)),
        out_shape=jax.ShapeDtypeStruct((2, N, F), jnp.float32),
    )(x, W1, degT)


def _mid_call(parts, degT, b1, W2):
    F = H // NC

    def body(p_ref, deg_ref, b1_ref, w2_ref, o_ref):
        agg = jnp.concatenate([p_ref[0], p_ref[1]], axis=1)
        nd = _norm_from(deg_ref[:, 1])
        ns = _norm_src_from(deg_ref)
        t = jnp.maximum(agg * nd[:, None] + b1_ref[...][None, :], 0.0) * ns[:, None]
        o_ref[...] = jnp.dot(t, w2_ref[...], preferred_element_type=jnp.float32)

    return pl.pallas_call(
        body,
        grid=(N // BN,),
        in_specs=[
            pl.BlockSpec((2, BN, F), lambda i: (0, i, 0)),
            pl.BlockSpec((BN, 2), lambda i: (i, 0)),
            pl.BlockSpec((H,), lambda i: (0,)),
            pl.BlockSpec((H, C), lambda i: (0, 0)),
        ],
        out_specs=pl.BlockSpec((BN, C), lambda i: (i, 0)),
        out_shape=jax.ShapeDtypeStruct((N, C), jnp.float32),
    )(parts, degT, b1, W2)


_PAD_DST = np.asarray(N + (np.arange(PADN) % 16), np.int32)
_PAD_SRC = np.asarray(np.arange(PADN), np.int32)


def kernel(x, edge_index, W1, b1, W2, b2):
    src = edge_index[0]
    dst = edge_index[1]
    dst_pd = jnp.concatenate([dst, jnp.asarray(_PAD_DST)]).reshape(EPAD // K, K)
    src_pg = jnp.concatenate([src, jnp.asarray(_PAD_SRC)]).reshape(EPAD // K, K)

    deg = _deg_call(src_pg, dst_pd)
    degT = deg.T
    h = _mm1_call(x, W1, degT)
    parts1 = _agg_split_call(h, src_pg, dst_pd)
    h2 = _mid_call(parts1, degT, b1, W2)
    parts2 = _agg2_call(h2, src_pg, dst_pd)
    nd = _norm_from(deg[1, :N])
    return (parts2[0, :N] + parts2[1, :N]) * nd[:, None] + b2[None, :]
